# baseline (speedup 1.0000x reference)
import functools

import jax
import jax.numpy as jnp
from jax import lax
from jax.experimental import pallas as pl
from jax.experimental.pallas import tpu as pltpu
from jax.experimental.pallas import tpu_sc as plsc

N = 10000
E = 320000
D = 128

NC = 2
NS = 16
NW = NC * NS
EW = E // NW
G = 80
SUB = 4
C = G // SUB
NGRP = EW // G
IB = 25
NIB = NGRP // IB

NPAD = 10240
RT = NPAD // NS
WC = 80


def _sc_body(x_hbm, src_hbm, dst_hbm, out_hbm,
             src_v, dst_v, rows0, rows1, gsem0, gsem1, ssem0, ssem1, acc):
    c = lax.axis_index("c")
    s = lax.axis_index("s")
    wid = s * NC + c

    @functools.partial(lax.fori_loop, 0, G * 8, init_val=None)
    def _(t, _):
        rows0[t // 8, pl.ds((t % 8) * 16, 16)] = jnp.zeros((16,), jnp.float32)
        return None

    tbase = s * RT
    zsrc = rows0.at[pl.ds(0, WC)]

    @functools.partial(lax.fori_loop, 0, RT // WC, init_val=None)
    def _(r, _):
        pltpu.sync_copy(zsrc, acc.at[pl.ds(tbase + r * WC, WC)])
        return None

    plsc.subcore_barrier()

    def fire_gathers(j, rows, gsem):
        for i in range(SUB):
            pltpu.async_copy(x_hbm.at[src_v.at[j * SUB + i]],
                             rows.at[pl.ds(i * C, C)], gsem)

    def drain_gathers(j, rows, gsem):
        for i in range(SUB):
            pltpu.make_async_copy(x_hbm.at[src_v.at[j * SUB + i]],
                                  rows.at[pl.ds(i * C, C)], gsem).wait()

    def fire_scatters(j, rows, ssem):
        for i in range(SUB):
            pltpu.async_copy(rows.at[pl.ds(i * C, C)],
                             acc.at[dst_v.at[j * SUB + i]], ssem, add=True)

    def drain_scatters(j, rows, ssem):
        for i in range(SUB):
            pltpu.make_async_copy(rows.at[pl.ds(i * C, C)],
                                  acc.at[dst_v.at[j * SUB + i]], ssem).wait()

    @functools.partial(lax.fori_loop, 0, NIB, init_val=None)
    def _(b, _):
        pltpu.sync_copy(src_hbm.at[wid, b], src_v)
        pltpu.sync_copy(dst_hbm.at[wid, b], dst_v)
        fire_gathers(0, rows0, gsem0)

        def step(j, rows_a, gsem_a, ssem_a, rows_b, gsem_b, ssem_b):
            drain_gathers(j, rows_a, gsem_a)
            fire_scatters(j, rows_a, ssem_a)

            @pl.when(j >= 1)
            def _():
                drain_scatters(j - 1, rows_b, ssem_b)

            @pl.when(j + 1 < IB)
            def _():
                fire_gathers(j + 1, rows_b, gsem_b)

        @functools.partial(lax.fori_loop, 0, IB, init_val=None)
        def _(j, _):
            @pl.when(j % 2 == 0)
            def _():
                step(j, rows0, gsem0, ssem0, rows1, gsem1, ssem1)

            @pl.when(j % 2 == 1)
            def _():
                step(j, rows1, gsem1, ssem1, rows0, gsem0, ssem0)

            return None

        if (IB - 1) % 2 == 0:
            drain_scatters(IB - 1, rows0, ssem0)
        else:
            drain_scatters(IB - 1, rows1, ssem1)
        return None

    plsc.subcore_barrier()

    @functools.partial(lax.fori_loop, 0, RT // WC, init_val=None)
    def _(r, _):
        pltpu.sync_copy(acc.at[pl.ds(tbase + r * WC, WC)], zsrc)
        pltpu.sync_copy(zsrc, out_hbm.at[c].at[pl.ds(tbase + r * WC, WC)])
        return None


_sc_segment_sum = functools.partial(
    pl.kernel,
    out_type=jax.ShapeDtypeStruct((NC, NPAD, D), jnp.float32),
    mesh=plsc.VectorSubcoreMesh(
        core_axis_name="c", subcore_axis_name="s",
        num_cores=NC, num_subcores=NS),
    scratch_types=[
        pltpu.VMEM((IB * SUB, C), jnp.int32),
        pltpu.VMEM((IB * SUB, C), jnp.int32),
        pltpu.VMEM((G, D), jnp.float32),
        pltpu.VMEM((G, D), jnp.float32),
        pltpu.SemaphoreType.DMA,
        pltpu.SemaphoreType.DMA,
        pltpu.SemaphoreType.DMA,
        pltpu.SemaphoreType.DMA,
        pltpu.VMEM_SHARED((NPAD, D), jnp.float32),
    ],
)(_sc_body)


BLK = 1000


def _mlp_body(relu_out, h_ref, p0_ref, p1_ref, wa_ref, ba_ref, wb_ref, bb_ref,
              o_ref):
    z = h_ref[...] + p0_ref[...] + p1_ref[...]
    a = jnp.dot(z, wa_ref[...], preferred_element_type=jnp.float32) + ba_ref[...]
    a = jnp.where(a > 0, a, a * 0.01)
    o = jnp.dot(a, wb_ref[...], preferred_element_type=jnp.float32) + bb_ref[...]
    if relu_out:
        o = jnp.where(o > 0, o, o * 0.01)
    o_ref[...] = o


def _mlp_tc(h, p, wa_t, ba, wb_t, bb, relu_out):
    row_spec = pl.BlockSpec((BLK, D), lambda i: (i, 0))
    part_spec = pl.BlockSpec((1, BLK, D), lambda i: (0, i, 0))
    full_spec = pl.BlockSpec((D, D), lambda i: (0, 0))
    bias_spec = pl.BlockSpec((1, D), lambda i: (0, 0))
    p0 = p[0:1]
    p1 = p[1:2]
    body = functools.partial(_mlp_body, relu_out)

    def wrapped(h_ref, p0_ref, p1_ref, wa_ref, ba_ref, wb_ref, bb_ref, o_ref):
        body(h_ref, p0_ref.at[0], p1_ref.at[0], wa_ref, ba_ref, wb_ref,
             bb_ref, o_ref)

    return pl.pallas_call(
        wrapped,
        grid=(N // BLK,),
        in_specs=[row_spec, part_spec, part_spec,
                  full_spec, bias_spec, full_spec, bias_spec],
        out_specs=row_spec,
        out_shape=jax.ShapeDtypeStruct((N, D), jnp.float32),
    )(h, p0, p1, wa_t, ba.reshape(1, D), wb_t, bb.reshape(1, D))


def kernel(x, edge_index, W1a, b1a, W1b, b1b, W2a, b2a, W2b, b2b,
           W3a, b3a, W3b, b3b):
    src = edge_index[0].reshape(NW, NIB, IB * SUB, C)
    dst = edge_index[1].reshape(NW, NIB, IB * SUB, C)

    w3b_t = jnp.zeros((D, D), jnp.float32).at[:, :2].set(W3b.T)
    b3b_p = jnp.zeros((D,), jnp.float32).at[:2].set(b3b)

    p = _sc_segment_sum(x, src, dst)
    h = _mlp_tc(x, p, W1a.T, b1a, W1b.T, b1b, relu_out=True)

    p = _sc_segment_sum(h, src, dst)
    h = _mlp_tc(h, p, W2a.T, b2a, W2b.T, b2b, relu_out=True)

    p = _sc_segment_sum(h, src, dst)
    out = _mlp_tc(h, p, W3a.T, b3a, w3b_t, b3b_p, relu_out=False)

    return out[:, :2]


# speedup vs baseline: 1.2653x; 1.2653x over previous
import functools

import jax
import jax.numpy as jnp
from jax import lax
from jax.experimental import pallas as pl
from jax.experimental.pallas import tpu as pltpu
from jax.experimental.pallas import tpu_sc as plsc

N = 10000
E = 320000
D = 128

NC = 2
NS = 16
NW = NC * NS
EW = E // NW
G = 80
SUB = 1
C = G // SUB
NGRP = EW // G
IB = 25
NIB = NGRP // IB

NPAD = 10240
RT = NPAD // NS
WC = 80


def _sc_body(x_hbm, src_hbm, dst_hbm, out_hbm,
             src_v, dst_v, rows0, rows1, gsem0, gsem1, ssem0, ssem1, acc):
    c = lax.axis_index("c")
    s = lax.axis_index("s")
    wid = s * NC + c

    @functools.partial(lax.fori_loop, 0, G * 8, init_val=None)
    def _(t, _):
        rows0[t // 8, pl.ds((t % 8) * 16, 16)] = jnp.zeros((16,), jnp.float32)
        return None

    tbase = s * RT
    zsrc = rows0.at[pl.ds(0, WC)]

    @functools.partial(lax.fori_loop, 0, RT // WC, init_val=None)
    def _(r, _):
        pltpu.sync_copy(zsrc, acc.at[pl.ds(tbase + r * WC, WC)])
        return None

    plsc.subcore_barrier()

    def fire_gathers(j, rows, gsem):
        for i in range(SUB):
            pltpu.async_copy(x_hbm.at[src_v.at[j * SUB + i]],
                             rows.at[pl.ds(i * C, C)], gsem)

    def drain_gathers(j, rows, gsem):
        for i in range(SUB):
            pltpu.make_async_copy(x_hbm.at[src_v.at[j * SUB + i]],
                                  rows.at[pl.ds(i * C, C)], gsem).wait()

    def fire_scatters(j, rows, ssem):
        for i in range(SUB):
            pltpu.async_copy(rows.at[pl.ds(i * C, C)],
                             acc.at[dst_v.at[j * SUB + i]], ssem, add=True)

    def drain_scatters(j, rows, ssem):
        for i in range(SUB):
            pltpu.make_async_copy(rows.at[pl.ds(i * C, C)],
                                  acc.at[dst_v.at[j * SUB + i]], ssem).wait()

    @functools.partial(lax.fori_loop, 0, NIB, init_val=None)
    def _(b, _):
        pltpu.sync_copy(src_hbm.at[wid, b], src_v)
        pltpu.sync_copy(dst_hbm.at[wid, b], dst_v)
        fire_gathers(0, rows0, gsem0)

        def step(j, rows_a, gsem_a, ssem_a, rows_b, gsem_b, ssem_b):
            @pl.when(j >= 1)
            def _():
                drain_scatters(j - 1, rows_b, ssem_b)

            @pl.when(j + 1 < IB)
            def _():
                fire_gathers(j + 1, rows_b, gsem_b)

            drain_gathers(j, rows_a, gsem_a)
            fire_scatters(j, rows_a, ssem_a)

        @functools.partial(lax.fori_loop, 0, IB, init_val=None)
        def _(j, _):
            @pl.when(j % 2 == 0)
            def _():
                step(j, rows0, gsem0, ssem0, rows1, gsem1, ssem1)

            @pl.when(j % 2 == 1)
            def _():
                step(j, rows1, gsem1, ssem1, rows0, gsem0, ssem0)

            return None

        if (IB - 1) % 2 == 0:
            drain_scatters(IB - 1, rows0, ssem0)
        else:
            drain_scatters(IB - 1, rows1, ssem1)
        return None

    plsc.subcore_barrier()

    @functools.partial(lax.fori_loop, 0, RT // WC, init_val=None)
    def _(r, _):
        pltpu.sync_copy(acc.at[pl.ds(tbase + r * WC, WC)], zsrc)
        pltpu.sync_copy(zsrc, out_hbm.at[c].at[pl.ds(tbase + r * WC, WC)])
        return None


_sc_segment_sum = functools.partial(
    pl.kernel,
    out_type=jax.ShapeDtypeStruct((NC, NPAD, D), jnp.float32),
    mesh=plsc.VectorSubcoreMesh(
        core_axis_name="c", subcore_axis_name="s",
        num_cores=NC, num_subcores=NS),
    scratch_types=[
        pltpu.VMEM((IB * SUB, C), jnp.int32),
        pltpu.VMEM((IB * SUB, C), jnp.int32),
        pltpu.VMEM((G, D), jnp.float32),
        pltpu.VMEM((G, D), jnp.float32),
        pltpu.SemaphoreType.DMA,
        pltpu.SemaphoreType.DMA,
        pltpu.SemaphoreType.DMA,
        pltpu.SemaphoreType.DMA,
        pltpu.VMEM_SHARED((NPAD, D), jnp.float32),
    ],
)(_sc_body)


BLK = 1000


def _mlp_body(relu_out, h_ref, p0_ref, p1_ref, wa_ref, ba_ref, wb_ref, bb_ref,
              o_ref):
    z = h_ref[...] + p0_ref[...] + p1_ref[...]
    a = jnp.dot(z, wa_ref[...], preferred_element_type=jnp.float32) + ba_ref[...]
    a = jnp.where(a > 0, a, a * 0.01)
    o = jnp.dot(a, wb_ref[...], preferred_element_type=jnp.float32) + bb_ref[...]
    if relu_out:
        o = jnp.where(o > 0, o, o * 0.01)
    o_ref[...] = o


def _mlp_tc(h, p, wa_t, ba, wb_t, bb, relu_out):
    row_spec = pl.BlockSpec((BLK, D), lambda i: (i, 0))
    part_spec = pl.BlockSpec((1, BLK, D), lambda i: (0, i, 0))
    full_spec = pl.BlockSpec((D, D), lambda i: (0, 0))
    bias_spec = pl.BlockSpec((1, D), lambda i: (0, 0))
    p0 = p[0:1]
    p1 = p[1:2]
    body = functools.partial(_mlp_body, relu_out)

    def wrapped(h_ref, p0_ref, p1_ref, wa_ref, ba_ref, wb_ref, bb_ref, o_ref):
        body(h_ref, p0_ref.at[0], p1_ref.at[0], wa_ref, ba_ref, wb_ref,
             bb_ref, o_ref)

    return pl.pallas_call(
        wrapped,
        grid=(N // BLK,),
        in_specs=[row_spec, part_spec, part_spec,
                  full_spec, bias_spec, full_spec, bias_spec],
        out_specs=row_spec,
        out_shape=jax.ShapeDtypeStruct((N, D), jnp.float32),
    )(h, p0, p1, wa_t, ba.reshape(1, D), wb_t, bb.reshape(1, D))


def kernel(x, edge_index, W1a, b1a, W1b, b1b, W2a, b2a, W2b, b2b,
           W3a, b3a, W3b, b3b):
    src = edge_index[0].reshape(NW, NIB, IB * SUB, C)
    dst = edge_index[1].reshape(NW, NIB, IB * SUB, C)

    w3b_t = jnp.zeros((D, D), jnp.float32).at[:, :2].set(W3b.T)
    b3b_p = jnp.zeros((D,), jnp.float32).at[:2].set(b3b)

    p = _sc_segment_sum(x, src, dst)
    h = _mlp_tc(x, p, W1a.T, b1a, W1b.T, b1b, relu_out=True)

    p = _sc_segment_sum(h, src, dst)
    h = _mlp_tc(h, p, W2a.T, b2a, W2b.T, b2b, relu_out=True)

    p = _sc_segment_sum(h, src, dst)
    out = _mlp_tc(h, p, W3a.T, b3a, w3b_t, b3b_p, relu_out=False)

    return out[:, :2]


# speedup vs baseline: 1.4013x; 1.1075x over previous
import functools

import jax
import jax.numpy as jnp
from jax import lax
from jax.experimental import pallas as pl
from jax.experimental.pallas import tpu as pltpu
from jax.experimental.pallas import tpu_sc as plsc

N = 10000
E = 320000
D = 128

NC = 2
NS = 16
NW = NC * NS
EW = E // NW
C = 40
NBUF = 4
NGRP = EW // C
IB = 50
NIB = NGRP // IB

NPAD = 10240
RT = NPAD // NS
WC = 40


def _sc_body(x_hbm, src_hbm, dst_hbm, out_hbm,
             src_v, dst_v, rows0, rows1, rows2, rows3,
             gsem0, gsem1, gsem2, gsem3, ssem0, ssem1, ssem2, ssem3, acc):
    c = lax.axis_index("c")
    s = lax.axis_index("s")
    wid = s * NC + c
    rows = (rows0, rows1, rows2, rows3)
    gsem = (gsem0, gsem1, gsem2, gsem3)
    ssem = (ssem0, ssem1, ssem2, ssem3)

    @functools.partial(lax.fori_loop, 0, C * 8, init_val=None)
    def _(t, _):
        rows0[t // 8, pl.ds((t % 8) * 16, 16)] = jnp.zeros((16,), jnp.float32)
        return None

    tbase = s * RT

    @functools.partial(lax.fori_loop, 0, RT // WC, init_val=None)
    def _(r, _):
        pltpu.sync_copy(rows0, acc.at[pl.ds(tbase + r * WC, WC)])
        return None

    plsc.subcore_barrier()

    def fire_gather(k, slot):
        pltpu.async_copy(x_hbm.at[src_v.at[k]], rows[slot], gsem[slot])

    def drain_gather(k, slot):
        pltpu.make_async_copy(
            x_hbm.at[src_v.at[k]], rows[slot], gsem[slot]).wait()

    def fire_scatter(k, slot):
        pltpu.async_copy(rows[slot], acc.at[dst_v.at[k]], ssem[slot],
                         add=True)

    def drain_scatter(k, slot):
        pltpu.make_async_copy(
            rows[slot], acc.at[dst_v.at[k]], ssem[slot]).wait()

    @functools.partial(lax.fori_loop, 0, NIB, init_val=None)
    def _(b, _):
        pltpu.sync_copy(src_hbm.at[wid, b], src_v)
        pltpu.sync_copy(dst_hbm.at[wid, b], dst_v)
        for k in range(NBUF - 1):
            fire_gather(k, k)

        def step(j, p):
            @pl.when(j >= 1)
            def _():
                drain_scatter(j - 1, (p + NBUF - 1) % NBUF)

            @pl.when(j + NBUF - 1 < IB)
            def _():
                fire_gather(j + NBUF - 1, (p + NBUF - 1) % NBUF)

            drain_gather(j, p)
            fire_scatter(j, p)

        @functools.partial(lax.fori_loop, 0, IB, init_val=None)
        def _(j, _):
            for p in range(NBUF):
                @pl.when(j % NBUF == p)
                def _(p=p):
                    step(j, p)

            return None

        drain_scatter(IB - 1, (IB - 1) % NBUF)
        return None

    plsc.subcore_barrier()

    @functools.partial(lax.fori_loop, 0, RT // WC, init_val=None)
    def _(r, _):
        pltpu.sync_copy(acc.at[pl.ds(tbase + r * WC, WC)], rows0)
        pltpu.sync_copy(rows0, out_hbm.at[c].at[pl.ds(tbase + r * WC, WC)])
        return None


_sc_segment_sum = functools.partial(
    pl.kernel,
    out_type=jax.ShapeDtypeStruct((NC, NPAD, D), jnp.float32),
    mesh=plsc.VectorSubcoreMesh(
        core_axis_name="c", subcore_axis_name="s",
        num_cores=NC, num_subcores=NS),
    scratch_types=(
        [pltpu.VMEM((IB, C), jnp.int32)] * 2
        + [pltpu.VMEM((C, D), jnp.float32)] * NBUF
        + [pltpu.SemaphoreType.DMA] * (2 * NBUF)
        + [pltpu.VMEM_SHARED((NPAD, D), jnp.float32)]
    ),
)(_sc_body)


BLK = 1000


def _mlp_body(relu_out, h_ref, p0_ref, p1_ref, wa_ref, ba_ref, wb_ref, bb_ref,
              o_ref):
    z = h_ref[...] + p0_ref[...] + p1_ref[...]
    a = jnp.dot(z, wa_ref[...], preferred_element_type=jnp.float32) + ba_ref[...]
    a = jnp.where(a > 0, a, a * 0.01)
    o = jnp.dot(a, wb_ref[...], preferred_element_type=jnp.float32) + bb_ref[...]
    if relu_out:
        o = jnp.where(o > 0, o, o * 0.01)
    o_ref[...] = o


def _mlp_tc(h, p, wa_t, ba, wb_t, bb, relu_out):
    row_spec = pl.BlockSpec((BLK, D), lambda i: (i, 0))
    part_spec = pl.BlockSpec((1, BLK, D), lambda i: (0, i, 0))
    full_spec = pl.BlockSpec((D, D), lambda i: (0, 0))
    bias_spec = pl.BlockSpec((1, D), lambda i: (0, 0))
    p0 = p[0:1]
    p1 = p[1:2]
    body = functools.partial(_mlp_body, relu_out)

    def wrapped(h_ref, p0_ref, p1_ref, wa_ref, ba_ref, wb_ref, bb_ref, o_ref):
        body(h_ref, p0_ref.at[0], p1_ref.at[0], wa_ref, ba_ref, wb_ref,
             bb_ref, o_ref)

    return pl.pallas_call(
        wrapped,
        grid=(N // BLK,),
        in_specs=[row_spec, part_spec, part_spec,
                  full_spec, bias_spec, full_spec, bias_spec],
        out_specs=row_spec,
        out_shape=jax.ShapeDtypeStruct((N, D), jnp.float32),
    )(h, p0, p1, wa_t, ba.reshape(1, D), wb_t, bb.reshape(1, D))


def kernel(x, edge_index, W1a, b1a, W1b, b1b, W2a, b2a, W2b, b2b,
           W3a, b3a, W3b, b3b):
    src = edge_index[0].reshape(NW, NIB, IB, C)
    dst = edge_index[1].reshape(NW, NIB, IB, C)

    w3b_t = jnp.zeros((D, D), jnp.float32).at[:, :2].set(W3b.T)
    b3b_p = jnp.zeros((D,), jnp.float32).at[:2].set(b3b)

    p = _sc_segment_sum(x, src, dst)
    h = _mlp_tc(x, p, W1a.T, b1a, W1b.T, b1b, relu_out=True)

    p = _sc_segment_sum(h, src, dst)
    h = _mlp_tc(h, p, W2a.T, b2a, W2b.T, b2b, relu_out=True)

    p = _sc_segment_sum(h, src, dst)
    out = _mlp_tc(h, p, W3a.T, b3a, w3b_t, b3b_p, relu_out=False)

    return out[:, :2]


# speedup vs baseline: 1.4515x; 1.0358x over previous
import functools

import jax
import jax.numpy as jnp
from jax import lax
from jax.experimental import pallas as pl
from jax.experimental.pallas import tpu as pltpu
from jax.experimental.pallas import tpu_sc as plsc

N = 10000
E = 320000
D = 128

NC = 2
NS = 16
NW = NC * NS
EW = E // NW
C = 40
NBUF = 4
NGRP = EW // C
IB = 50
NIB = NGRP // IB

NPAD = 10240
RT = NPAD // NS
WC = 40


def _sc_body(x_hbm, src_hbm, dst_hbm, out_hbm,
             src_v, dst_v, rows0, rows1, rows2, rows3,
             gsem0, gsem1, gsem2, gsem3, ssem0, ssem1, ssem2, ssem3, acc):
    c = lax.axis_index("c")
    s = lax.axis_index("s")
    wid = s * NC + c
    rows = (rows0, rows1, rows2, rows3)
    gsem = (gsem0, gsem1, gsem2, gsem3)
    ssem = (ssem0, ssem1, ssem2, ssem3)

    @functools.partial(lax.fori_loop, 0, C * 8, init_val=None)
    def _(t, _):
        rows0[t // 8, pl.ds((t % 8) * 16, 16)] = jnp.zeros((16,), jnp.float32)
        return None

    tbase = s * RT

    @functools.partial(lax.fori_loop, 0, RT // WC, init_val=None)
    def _(r, _):
        pltpu.sync_copy(rows0, acc.at[pl.ds(tbase + r * WC, WC)])
        return None

    plsc.subcore_barrier()

    def fire_gather(k, slot):
        pltpu.async_copy(x_hbm.at[src_v.at[k]], rows[slot], gsem[slot])

    def drain_gather(k, slot):
        pltpu.make_async_copy(
            x_hbm.at[src_v.at[k]], rows[slot], gsem[slot]).wait()

    def fire_scatter(k, slot):
        pltpu.async_copy(rows[slot], acc.at[dst_v.at[k]], ssem[slot],
                         add=True)

    def drain_scatter(k, slot):
        pltpu.make_async_copy(
            rows[slot], acc.at[dst_v.at[k]], ssem[slot]).wait()

    @functools.partial(lax.fori_loop, 0, NIB, init_val=None)
    def _(b, _):
        pltpu.sync_copy(src_hbm.at[wid, b], src_v)
        pltpu.sync_copy(dst_hbm.at[wid, b], dst_v)
        for k in range(NBUF - 1):
            fire_gather(k, k)

        def step(j, p):
            @pl.when(j >= 1)
            def _():
                drain_scatter(j - 1, (p + NBUF - 1) % NBUF)

            @pl.when(j + NBUF - 1 < IB)
            def _():
                fire_gather(j + NBUF - 1, (p + NBUF - 1) % NBUF)

            drain_gather(j, p)
            fire_scatter(j, p)

        @functools.partial(lax.fori_loop, 0, IB, init_val=None)
        def _(j, _):
            for p in range(NBUF):
                @pl.when(j % NBUF == p)
                def _(p=p):
                    step(j, p)

            return None

        drain_scatter(IB - 1, (IB - 1) % NBUF)
        return None

    plsc.subcore_barrier()

    @functools.partial(lax.fori_loop, 0, RT // WC, init_val=None)
    def _(r, _):
        pltpu.sync_copy(acc.at[pl.ds(tbase + r * WC, WC)],
                        out_hbm.at[c].at[pl.ds(tbase + r * WC, WC)])
        return None


_sc_segment_sum = functools.partial(
    pl.kernel,
    out_type=jax.ShapeDtypeStruct((NC, NPAD, D), jnp.float32),
    mesh=plsc.VectorSubcoreMesh(
        core_axis_name="c", subcore_axis_name="s",
        num_cores=NC, num_subcores=NS),
    scratch_types=(
        [pltpu.VMEM((IB, C), jnp.int32)] * 2
        + [pltpu.VMEM((C, D), jnp.float32)] * NBUF
        + [pltpu.SemaphoreType.DMA] * (2 * NBUF)
        + [pltpu.VMEM_SHARED((NPAD, D), jnp.float32)]
    ),
)(_sc_body)


BLK = 1000


def _mlp_body(relu_out, h_ref, p_ref, wa_ref, ba_ref, wb_ref, bb_ref, o_ref):
    z = h_ref[...] + p_ref[0] + p_ref[1]
    a = jnp.dot(z, wa_ref[...], preferred_element_type=jnp.float32) + ba_ref[...]
    a = jnp.where(a > 0, a, a * 0.01)
    o = jnp.dot(a, wb_ref[...], preferred_element_type=jnp.float32) + bb_ref[...]
    if relu_out:
        o = jnp.where(o > 0, o, o * 0.01)
    o_ref[...] = o


def _mlp_tc(h, p, wa_t, ba, wb_t, bb, relu_out):
    row_spec = pl.BlockSpec((BLK, D), lambda i: (i, 0))
    part_spec = pl.BlockSpec((2, BLK, D), lambda i: (0, i, 0))
    full_spec = pl.BlockSpec((D, D), lambda i: (0, 0))
    bias_spec = pl.BlockSpec((1, D), lambda i: (0, 0))
    return pl.pallas_call(
        functools.partial(_mlp_body, relu_out),
        grid=(N // BLK,),
        in_specs=[row_spec, part_spec,
                  full_spec, bias_spec, full_spec, bias_spec],
        out_specs=row_spec,
        out_shape=jax.ShapeDtypeStruct((N, D), jnp.float32),
    )(h, p, wa_t, ba.reshape(1, D), wb_t, bb.reshape(1, D))


def kernel(x, edge_index, W1a, b1a, W1b, b1b, W2a, b2a, W2b, b2b,
           W3a, b3a, W3b, b3b):
    src = edge_index[0].reshape(NW, NIB, IB, C)
    dst = edge_index[1].reshape(NW, NIB, IB, C)

    w3b_t = jnp.zeros((D, D), jnp.float32).at[:, :2].set(W3b.T)
    b3b_p = jnp.zeros((D,), jnp.float32).at[:2].set(b3b)

    p = _sc_segment_sum(x, src, dst)
    h = _mlp_tc(x, p, W1a.T, b1a, W1b.T, b1b, relu_out=True)

    p = _sc_segment_sum(h, src, dst)
    h = _mlp_tc(h, p, W2a.T, b2a, W2b.T, b2b, relu_out=True)

    p = _sc_segment_sum(h, src, dst)
    out = _mlp_tc(h, p, W3a.T, b3a, w3b_t, b3b_p, relu_out=False)

    return out[:, :2]


# speedup vs baseline: 1.5090x; 1.0396x over previous
import functools

import jax
import jax.numpy as jnp
from jax import lax
from jax.experimental import pallas as pl
from jax.experimental.pallas import tpu as pltpu
from jax.experimental.pallas import tpu_sc as plsc

N = 10000
E = 320000
D = 128

NC = 2
NS = 16
NW = NC * NS
EW = E // NW
C = 40
NBUF = 4
NGRP = EW // C
IB = 50
NIB = NGRP // IB

NPAD = 10240
RT = NPAD // NS
WC = 40


def _sc_body(x_hbm, src_hbm, dst_hbm, out_hbm,
             src_v, dst_v, rows0, rows1, rows2, rows3,
             gsem0, gsem1, gsem2, gsem3, ssem0, ssem1, ssem2, ssem3, acc):
    c = lax.axis_index("c")
    s = lax.axis_index("s")
    wid = s * NC + c
    rows = (rows0, rows1, rows2, rows3)
    gsem = (gsem0, gsem1, gsem2, gsem3)
    ssem = (ssem0, ssem1, ssem2, ssem3)

    tbase = s * RT

    def fire_gather(k, slot):
        pltpu.async_copy(x_hbm.at[src_v.at[k]], rows[slot], gsem[slot])

    def drain_gather(k, slot):
        pltpu.make_async_copy(
            x_hbm.at[src_v.at[k]], rows[slot], gsem[slot]).wait()

    def fire_scatter(k, slot):
        pltpu.async_copy(rows[slot], acc.at[dst_v.at[k]], ssem[slot],
                         add=True)

    def drain_scatter(k, slot):
        pltpu.make_async_copy(
            rows[slot], acc.at[dst_v.at[k]], ssem[slot]).wait()

    pltpu.sync_copy(src_hbm.at[wid, 0], src_v)
    pltpu.sync_copy(dst_hbm.at[wid, 0], dst_v)
    for k in range(NBUF - 1):
        fire_gather(k, k)

    @functools.partial(lax.fori_loop, 0, C * 8, init_val=None)
    def _(t, _):
        rows3[t // 8, pl.ds((t % 8) * 16, 16)] = jnp.zeros((16,), jnp.float32)
        return None

    @functools.partial(lax.fori_loop, 0, RT // WC, init_val=None)
    def _(r, _):
        pltpu.sync_copy(rows3, acc.at[pl.ds(tbase + r * WC, WC)])
        return None

    plsc.subcore_barrier()

    @functools.partial(lax.fori_loop, 0, NIB, init_val=None)
    def _(b, _):
        @pl.when(b >= 1)
        def _():
            pltpu.sync_copy(src_hbm.at[wid, b], src_v)
            pltpu.sync_copy(dst_hbm.at[wid, b], dst_v)
            for k in range(NBUF - 1):
                fire_gather(k, k)

        def step(j, p):
            @pl.when(j >= 1)
            def _():
                drain_scatter(j - 1, (p + NBUF - 1) % NBUF)

            @pl.when(j + NBUF - 1 < IB)
            def _():
                fire_gather(j + NBUF - 1, (p + NBUF - 1) % NBUF)

            drain_gather(j, p)
            fire_scatter(j, p)

        @functools.partial(lax.fori_loop, 0, IB, init_val=None)
        def _(j, _):
            for p in range(NBUF):
                @pl.when(j % NBUF == p)
                def _(p=p):
                    step(j, p)

            return None

        drain_scatter(IB - 1, (IB - 1) % NBUF)
        return None

    plsc.subcore_barrier()

    def wr_copy(r):
        return pltpu.make_async_copy(
            acc.at[pl.ds(tbase + r * WC, WC)],
            out_hbm.at[c].at[pl.ds(tbase + r * WC, WC)], gsem0)

    @functools.partial(lax.fori_loop, 0, RT // WC, init_val=None)
    def _(r, _):
        wr_copy(r).start()
        return None

    @functools.partial(lax.fori_loop, 0, RT // WC, init_val=None)
    def _(r, _):
        wr_copy(r).wait()
        return None


_sc_segment_sum = functools.partial(
    pl.kernel,
    out_type=jax.ShapeDtypeStruct((NC, NPAD, D), jnp.float32),
    mesh=plsc.VectorSubcoreMesh(
        core_axis_name="c", subcore_axis_name="s",
        num_cores=NC, num_subcores=NS),
    scratch_types=(
        [pltpu.VMEM((IB, C), jnp.int32)] * 2
        + [pltpu.VMEM((C, D), jnp.float32)] * NBUF
        + [pltpu.SemaphoreType.DMA] * (2 * NBUF)
        + [pltpu.VMEM_SHARED((NPAD, D), jnp.float32)]
    ),
)(_sc_body)


BLK = 1000


def _mlp_body(relu_out, h_ref, p_ref, wa_ref, ba_ref, wb_ref, bb_ref, o_ref):
    z = h_ref[...] + p_ref[0] + p_ref[1]
    a = jnp.dot(z, wa_ref[...], preferred_element_type=jnp.float32) + ba_ref[...]
    a = jnp.where(a > 0, a, a * 0.01)
    o = jnp.dot(a, wb_ref[...], preferred_element_type=jnp.float32) + bb_ref[...]
    if relu_out:
        o = jnp.where(o > 0, o, o * 0.01)
    o_ref[...] = o


def _mlp_tc(h, p, wa_t, ba, wb_t, bb, relu_out):
    row_spec = pl.BlockSpec((BLK, D), lambda i: (i, 0))
    part_spec = pl.BlockSpec((2, BLK, D), lambda i: (0, i, 0))
    full_spec = pl.BlockSpec((D, D), lambda i: (0, 0))
    bias_spec = pl.BlockSpec((1, D), lambda i: (0, 0))
    return pl.pallas_call(
        functools.partial(_mlp_body, relu_out),
        grid=(N // BLK,),
        in_specs=[row_spec, part_spec,
                  full_spec, bias_spec, full_spec, bias_spec],
        out_specs=row_spec,
        out_shape=jax.ShapeDtypeStruct((N, D), jnp.float32),
    )(h, p, wa_t, ba.reshape(1, D), wb_t, bb.reshape(1, D))


def kernel(x, edge_index, W1a, b1a, W1b, b1b, W2a, b2a, W2b, b2b,
           W3a, b3a, W3b, b3b):
    src = edge_index[0].reshape(NW, NIB, IB, C)
    dst = edge_index[1].reshape(NW, NIB, IB, C)

    w3b_t = jnp.zeros((D, D), jnp.float32).at[:, :2].set(W3b.T)
    b3b_p = jnp.zeros((D,), jnp.float32).at[:2].set(b3b)

    p = _sc_segment_sum(x, src, dst)
    h = _mlp_tc(x, p, W1a.T, b1a, W1b.T, b1b, relu_out=True)

    p = _sc_segment_sum(h, src, dst)
    h = _mlp_tc(h, p, W2a.T, b2a, W2b.T, b2b, relu_out=True)

    p = _sc_segment_sum(h, src, dst)
    out = _mlp_tc(h, p, W3a.T, b3a, w3b_t, b3b_p, relu_out=False)

    return out[:, :2]
